# Initial kernel scaffold; baseline (speedup 1.0000x reference)
import jax
import jax.numpy as jnp
from jax.experimental import pallas as pl


def kernel(x, edge_index, edge_attr, ee1, ee2, W1, b1, W2, b2):
    raise NotImplementedError("write your pallas kernel here")


# speedup vs baseline: 1.9147x; 1.9147x over previous
import functools
import jax
import jax.numpy as jnp
from jax import lax
from jax.experimental import pallas as pl
from jax.experimental.pallas import tpu as pltpu
from jax.experimental.pallas import tpu_sc as plsc

NC = 2
NS = 16
NW = NC * NS
B = 128
CPT = 80
EPT = B * CPT
EPAD = NW * EPT
TROWS = 32


def _tbl_body(e1_ref, e2_ref, out_ref):
  c = lax.broadcasted_iota(jnp.int32, (TROWS, 1), 0)
  i6 = lax.broadcasted_iota(jnp.int32, (1, 6), 1)
  i3 = lax.broadcasted_iota(jnp.int32, (1, 3), 1)
  s1 = (c // 3 == i6).astype(jnp.float32)
  s2 = ((c % 3 == i3) & (c < 18)).astype(jnp.float32)
  out_ref[...] = (jnp.dot(s1, e1_ref[...], preferred_element_type=jnp.float32)
                  + jnp.dot(s2, e2_ref[...], preferred_element_type=jnp.float32))


def _tc_table(ee1, ee2):
  d = ee1.shape[1]
  return pl.pallas_call(
      _tbl_body,
      out_shape=jax.ShapeDtypeStruct((TROWS, d), jnp.float32),
  )(ee1, ee2)


def _sc_scatter(x, srcp, dstp, combop, tbl, n_pad):
  D = x.shape[1]
  rpt = n_pad // NS
  mesh = plsc.VectorSubcoreMesh(core_axis_name="c", subcore_axis_name="s")

  @functools.partial(
      pl.kernel,
      out_type=[
          jax.ShapeDtypeStruct((NC, n_pad, D), jnp.float32),
      ],
      mesh=mesh,
      scratch_types=[
          pltpu.VMEM((B,), jnp.int32),
          pltpu.VMEM((B,), jnp.int32),
          pltpu.VMEM((B,), jnp.int32),
          pltpu.VMEM((B, 128), jnp.float32),
          pltpu.VMEM_SHARED((n_pad, 128), jnp.float32),
          pltpu.SemaphoreType.DMA,
      ],
  )
  def k(x_hbm, src_hbm, dst_hbm, combo_hbm, tbl_hbm, aggr_out,
        src_v, dst_v, combo_v, rows_v, aggr_sh, sem):
    c = lax.axis_index("c")
    s = lax.axis_index("s")
    wid = c * NS + s

    def zero_rows(i, carry):
      def zcol(j, carry2):
        rows_v[i, pl.ds(j * 16, 16)] = jnp.zeros((16,), jnp.float32)
        return carry2
      return lax.fori_loop(0, 128 // 16, zcol, carry)
    lax.fori_loop(0, B, zero_rows, 0)

    def zero_shared(j, carry):
      pltpu.sync_copy(rows_v, aggr_sh.at[pl.ds(s * rpt + j * B, B)])
      return carry
    lax.fori_loop(0, rpt // B, zero_shared, 0)

    plsc.subcore_barrier()

    def block(g, carry):
      base = wid * CPT + g
      pltpu.sync_copy(src_hbm.at[base], src_v)
      pltpu.sync_copy(dst_hbm.at[base], dst_v)
      pltpu.sync_copy(combo_hbm.at[base], combo_v)
      pltpu.async_copy(x_hbm.at[src_v], rows_v, sem).wait()
      pltpu.sync_copy(rows_v, aggr_sh.at[dst_v], add=True)
      pltpu.async_copy(tbl_hbm.at[combo_v], rows_v, sem).wait()
      pltpu.sync_copy(rows_v, aggr_sh.at[dst_v], add=True)
      return carry
    lax.fori_loop(0, CPT, block, 0)

    plsc.subcore_barrier()

    pltpu.sync_copy(aggr_sh.at[pl.ds(s * rpt, rpt)],
                    aggr_out.at[c, pl.ds(s * rpt, rpt)])

  return k(x, srcp, dstp, combop, tbl)


def _mlp_body(p0, p1, x_ref, tbl, w1, b1, w2, b2, out_ref):
  self_row = tbl[12:13, :]
  aggr = p0[...] + p1[...] + x_ref[...] + self_row
  h = jnp.dot(aggr, w1[...], preferred_element_type=jnp.float32) + b1[...]
  h = jnp.maximum(h, 0.0)
  out_ref[...] = jnp.dot(h, w2[...], preferred_element_type=jnp.float32) + b2[...]


def _tc_mlp(p0, p1, x, tbl, w1, b1, w2, b2):
  n, d = x.shape
  r = 1000
  blk = lambda rr, cc: pl.BlockSpec((rr, cc), lambda i: (i, 0))
  full = lambda rr, cc: pl.BlockSpec((rr, cc), lambda i: (0, 0))
  return pl.pallas_call(
      _mlp_body,
      grid=(n // r,),
      in_specs=[
          blk(r, d), blk(r, d), blk(r, d),
          full(TROWS, d), full(d, w1.shape[1]), full(1, b1.shape[1]),
          full(w2.shape[0], d), full(1, d),
      ],
      out_specs=blk(r, d),
      out_shape=jax.ShapeDtypeStruct((n, d), jnp.float32),
  )(p0, p1, x, tbl, w1, b1, w2, b2)


@jax.jit
def kernel(x, edge_index, edge_attr, ee1, ee2, W1, b1, W2, b2):
  n, d = x.shape
  e = edge_index.shape[1]
  n_pad = 10240
  pad = EPAD - e

  src = jnp.pad(edge_index[0], (0, pad)).reshape(NW * CPT, B)
  dst = jnp.pad(edge_index[1], (0, pad), constant_values=n).reshape(NW * CPT, B)
  combo = jnp.pad(edge_attr[:, 0] * 3 + edge_attr[:, 1],
                  (0, pad)).reshape(NW * CPT, B)

  tbl = _tc_table(ee1, ee2)
  (aggr_p,) = _sc_scatter(x, src, dst, combo, tbl, n_pad)

  return _tc_mlp(aggr_p[0, :n], aggr_p[1, :n], x, tbl,
                 W1, b1.reshape(1, -1), W2, b2.reshape(1, -1))


# speedup vs baseline: 2.2604x; 1.1806x over previous
import functools
import jax
import jax.numpy as jnp
from jax import lax
from jax.experimental import pallas as pl
from jax.experimental.pallas import tpu as pltpu
from jax.experimental.pallas import tpu_sc as plsc

NC = 2
NS = 16
NW = NC * NS
B = 128
CPT = 80
EPT = B * CPT
EPAD = NW * EPT
TROWS = 32


def _tbl_body(e1_ref, e2_ref, out_ref):
  c = lax.broadcasted_iota(jnp.int32, (TROWS, 1), 0)
  i6 = lax.broadcasted_iota(jnp.int32, (1, 6), 1)
  i3 = lax.broadcasted_iota(jnp.int32, (1, 3), 1)
  s1 = (c // 3 == i6).astype(jnp.float32)
  s2 = ((c % 3 == i3) & (c < 18)).astype(jnp.float32)
  out_ref[...] = (jnp.dot(s1, e1_ref[...], preferred_element_type=jnp.float32)
                  + jnp.dot(s2, e2_ref[...], preferred_element_type=jnp.float32))


def _tc_table(ee1, ee2):
  d = ee1.shape[1]
  return pl.pallas_call(
      _tbl_body,
      out_shape=jax.ShapeDtypeStruct((TROWS, d), jnp.float32),
  )(ee1, ee2)


def _sc_scatter(x, srcp, dstp, combop, tbl, n_pad):
  D = x.shape[1]
  rpt = n_pad // NS
  mesh = plsc.VectorSubcoreMesh(core_axis_name="c", subcore_axis_name="s")

  @functools.partial(
      pl.kernel,
      out_type=[
          jax.ShapeDtypeStruct((NC, n_pad, D), jnp.float32),
      ],
      mesh=mesh,
      scratch_types=[
          pltpu.VMEM((B,), jnp.int32),
          pltpu.VMEM((B,), jnp.int32),
          pltpu.VMEM((B,), jnp.int32),
          pltpu.VMEM((B,), jnp.int32),
          pltpu.VMEM((B,), jnp.int32),
          pltpu.VMEM((B,), jnp.int32),
          pltpu.VMEM((B, 128), jnp.float32),
          pltpu.VMEM((B, 128), jnp.float32),
          pltpu.VMEM_SHARED((n_pad, 128), jnp.float32),
          pltpu.SemaphoreType.DMA,
          pltpu.SemaphoreType.DMA,
          pltpu.SemaphoreType.DMA,
          pltpu.SemaphoreType.DMA,
      ],
  )
  def k(x_hbm, src_hbm, dst_hbm, combo_hbm, tbl_hbm, aggr_out,
        src_a, src_b, dst_a, dst_b, combo_a, combo_b, rx_v, rt_v, aggr_sh,
        gsem_x, gsem_t, ssem_x, ssem_t):
    c = lax.axis_index("c")
    s = lax.axis_index("s")
    wid = c * NS + s

    def zero_rows(i, carry):
      def zcol(j, carry2):
        rx_v[i, pl.ds(j * 16, 16)] = jnp.zeros((16,), jnp.float32)
        return carry2
      return lax.fori_loop(0, 128 // 16, zcol, carry)
    lax.fori_loop(0, B, zero_rows, 0)

    def zero_shared(j, carry):
      pltpu.sync_copy(rx_v, aggr_sh.at[pl.ds(s * rpt + j * B, B)])
      return carry
    lax.fori_loop(0, rpt // B, zero_shared, 0)

    plsc.subcore_barrier()

    def stage_idx(g, sv, dv, cv):
      base = wid * CPT + g
      pltpu.sync_copy(src_hbm.at[base], sv)
      pltpu.sync_copy(dst_hbm.at[base], dv)
      pltpu.sync_copy(combo_hbm.at[base], cv)

    stage_idx(0, src_a, dst_a, combo_a)
    pltpu.async_copy(x_hbm.at[src_a], rx_v, gsem_x)
    pltpu.async_copy(tbl_hbm.at[combo_a], rt_v, gsem_t)

    def pair(p, carry):
      for q, (sv, dv, cv, sv2, dv2, cv2) in enumerate([
          (src_a, dst_a, combo_a, src_b, dst_b, combo_b),
          (src_b, dst_b, combo_b, src_a, dst_a, combo_a)]):
        g = 2 * p + q
        pltpu.make_async_copy(x_hbm.at[sv], rx_v, gsem_x).wait()
        pltpu.make_async_copy(tbl_hbm.at[cv], rt_v, gsem_t).wait()
        pltpu.async_copy(rx_v, aggr_sh.at[dv], ssem_x, add=True)
        pltpu.async_copy(rt_v, aggr_sh.at[dv], ssem_t, add=True)
        @pl.when(g + 1 < CPT)
        def _():
          stage_idx(g + 1, sv2, dv2, cv2)
        pltpu.make_async_copy(rx_v, aggr_sh.at[dv], ssem_x).wait()
        pltpu.make_async_copy(rt_v, aggr_sh.at[dv], ssem_t).wait()
        @pl.when(g + 1 < CPT)
        def _():
          pltpu.async_copy(x_hbm.at[sv2], rx_v, gsem_x)
          pltpu.async_copy(tbl_hbm.at[cv2], rt_v, gsem_t)
      return carry
    lax.fori_loop(0, CPT // 2, pair, 0)

    plsc.subcore_barrier()

    pltpu.sync_copy(aggr_sh.at[pl.ds(s * rpt, rpt)],
                    aggr_out.at[c, pl.ds(s * rpt, rpt)])

  return k(x, srcp, dstp, combop, tbl)


def _mlp_body(p0, p1, x_ref, tbl, w1, b1, w2, b2, out_ref):
  self_row = tbl[12:13, :]
  aggr = p0[...] + p1[...] + x_ref[...] + self_row
  h = jnp.dot(aggr, w1[...], preferred_element_type=jnp.float32) + b1[...]
  h = jnp.maximum(h, 0.0)
  out_ref[...] = jnp.dot(h, w2[...], preferred_element_type=jnp.float32) + b2[...]


def _tc_mlp(p0, p1, x, tbl, w1, b1, w2, b2):
  n, d = x.shape
  r = 1000
  blk = lambda rr, cc: pl.BlockSpec((rr, cc), lambda i: (i, 0))
  full = lambda rr, cc: pl.BlockSpec((rr, cc), lambda i: (0, 0))
  return pl.pallas_call(
      _mlp_body,
      grid=(n // r,),
      in_specs=[
          blk(r, d), blk(r, d), blk(r, d),
          full(TROWS, d), full(d, w1.shape[1]), full(1, b1.shape[1]),
          full(w2.shape[0], d), full(1, d),
      ],
      out_specs=blk(r, d),
      out_shape=jax.ShapeDtypeStruct((n, d), jnp.float32),
  )(p0, p1, x, tbl, w1, b1, w2, b2)


@jax.jit
def kernel(x, edge_index, edge_attr, ee1, ee2, W1, b1, W2, b2):
  n, d = x.shape
  e = edge_index.shape[1]
  n_pad = 10240
  pad = EPAD - e

  src = jnp.pad(edge_index[0], (0, pad)).reshape(NW * CPT, B)
  dst = jnp.pad(edge_index[1], (0, pad), constant_values=n).reshape(NW * CPT, B)
  combo = jnp.pad(edge_attr[:, 0] * 3 + edge_attr[:, 1],
                  (0, pad)).reshape(NW * CPT, B)

  tbl = _tc_table(ee1, ee2)
  (aggr_p,) = _sc_scatter(x, src, dst, combo, tbl, n_pad)

  return _tc_mlp(aggr_p[0, :n], aggr_p[1, :n], x, tbl,
                 W1, b1.reshape(1, -1), W2, b2.reshape(1, -1))


# speedup vs baseline: 6.1872x; 2.7372x over previous
import functools
import jax
import jax.numpy as jnp
from jax import lax
from jax.experimental import pallas as pl
from jax.experimental.pallas import tpu as pltpu
from jax.experimental.pallas import tpu_sc as plsc

NC = 2
NS = 16
NW = NC * NS
B = 128
CPT = 80
EPT = B * CPT
EPAD = NW * EPT
TROWS = 32


def _tbl_body(e1_ref, e2_ref, out_ref):
  c = lax.broadcasted_iota(jnp.int32, (TROWS, 1), 0)
  i6 = lax.broadcasted_iota(jnp.int32, (1, 6), 1)
  i3 = lax.broadcasted_iota(jnp.int32, (1, 3), 1)
  s1 = (c // 3 == i6).astype(jnp.float32)
  s2 = ((c % 3 == i3) & (c < 18)).astype(jnp.float32)
  out_ref[...] = (jnp.dot(s1, e1_ref[...], preferred_element_type=jnp.float32)
                  + jnp.dot(s2, e2_ref[...], preferred_element_type=jnp.float32))


def _tc_table(ee1, ee2):
  d = ee1.shape[1]
  return pl.pallas_call(
      _tbl_body,
      out_shape=jax.ShapeDtypeStruct((TROWS, d), jnp.float32),
  )(ee1, ee2)


def _sc_scatter(x, srcp, dstp, combop, tbl, n_pad):
  D = x.shape[1]
  rpt = n_pad // NS
  mesh = plsc.VectorSubcoreMesh(core_axis_name="c", subcore_axis_name="s")

  @functools.partial(
      pl.kernel,
      out_type=[
          jax.ShapeDtypeStruct((NC, n_pad, D), jnp.float32),
      ],
      mesh=mesh,
      scratch_types=[
          pltpu.VMEM((B,), jnp.int32),
          pltpu.VMEM((B,), jnp.int32),
          pltpu.VMEM((B,), jnp.int32),
          pltpu.VMEM((B,), jnp.int32),
          pltpu.VMEM((B,), jnp.int32),
          pltpu.VMEM((B,), jnp.int32),
          pltpu.VMEM((B, 128), jnp.float32),
          pltpu.VMEM((B, 128), jnp.float32),
          pltpu.VMEM_SHARED((n_pad, 128), jnp.float32),
          pltpu.SemaphoreType.DMA,
          pltpu.SemaphoreType.DMA,
          pltpu.SemaphoreType.DMA,
          pltpu.SemaphoreType.DMA,
      ],
  )
  def k(x_hbm, src_hbm, dst_hbm, combo_hbm, tbl_hbm, aggr_out,
        src_a, src_b, dst_a, dst_b, combo_a, combo_b, rx_v, rt_v, aggr_sh,
        gsem_x, gsem_t, ssem_x, ssem_t):
    c = lax.axis_index("c")
    s = lax.axis_index("s")
    wid = c * NS + s

    def zero_rows(i, carry):
      def zcol(j, carry2):
        rx_v[i, pl.ds(j * 16, 16)] = jnp.zeros((16,), jnp.float32)
        return carry2
      return lax.fori_loop(0, 128 // 16, zcol, carry)
    lax.fori_loop(0, B, zero_rows, 0)

    def zero_shared(j, carry):
      pltpu.sync_copy(rx_v, aggr_sh.at[pl.ds(s * rpt + j * B, B)])
      return carry
    lax.fori_loop(0, rpt // B, zero_shared, 0)

    plsc.subcore_barrier()

    def stage_idx(g, sv, dv, cv):
      base = wid * CPT + g
      pltpu.sync_copy(src_hbm.at[base], sv)
      pltpu.sync_copy(dst_hbm.at[base], dv)
      pltpu.sync_copy(combo_hbm.at[base], cv)

    stage_idx(0, src_a, dst_a, combo_a)
    pltpu.async_copy(x_hbm.at[src_a], rx_v, gsem_x)
    pltpu.async_copy(tbl_hbm.at[combo_a], rt_v, gsem_t)

    def pair(p, carry):
      for q, (sv, dv, cv, sv2, dv2, cv2) in enumerate([
          (src_a, dst_a, combo_a, src_b, dst_b, combo_b),
          (src_b, dst_b, combo_b, src_a, dst_a, combo_a)]):
        g = 2 * p + q
        pltpu.make_async_copy(x_hbm.at[sv], rx_v, gsem_x).wait()
        pltpu.make_async_copy(tbl_hbm.at[cv], rt_v, gsem_t).wait()
        pltpu.async_copy(rx_v, aggr_sh.at[dv], ssem_x, add=True)
        pltpu.async_copy(rt_v, aggr_sh.at[dv], ssem_t, add=True)
        @pl.when(g + 1 < CPT)
        def _():
          stage_idx(g + 1, sv2, dv2, cv2)
        pltpu.make_async_copy(rx_v, aggr_sh.at[dv], ssem_x).wait()
        pltpu.make_async_copy(rt_v, aggr_sh.at[dv], ssem_t).wait()
        @pl.when(g + 1 < CPT)
        def _():
          pltpu.async_copy(x_hbm.at[sv2], rx_v, gsem_x)
          pltpu.async_copy(tbl_hbm.at[cv2], rt_v, gsem_t)
      return carry
    lax.fori_loop(0, CPT // 2, pair, 0)

    plsc.subcore_barrier()

    pltpu.sync_copy(aggr_sh.at[pl.ds(s * rpt, rpt)],
                    aggr_out.at[c, pl.ds(s * rpt, rpt)])

  return k(x, srcp, dstp, combop, tbl)


def _mlp_body(p0, p1, x_ref, tbl, w1, b1, w2, b2, out_ref):
  self_row = tbl[12:13, :]
  aggr = p0[...] + p1[...] + x_ref[...] + self_row
  h = jnp.dot(aggr, w1[...], preferred_element_type=jnp.float32) + b1[...]
  h = jnp.maximum(h, 0.0)
  out_ref[...] = jnp.dot(h, w2[...], preferred_element_type=jnp.float32) + b2[...]


def _tc_mlp(p0, p1, x, tbl, w1, b1, w2, b2):
  n, d = x.shape
  r = 1000
  blk = lambda rr, cc: pl.BlockSpec((rr, cc), lambda i: (i, 0))
  full = lambda rr, cc: pl.BlockSpec((rr, cc), lambda i: (0, 0))
  return pl.pallas_call(
      _mlp_body,
      grid=(n // r,),
      in_specs=[
          blk(r, d), blk(r, d), blk(r, d),
          full(TROWS, d), full(d, w1.shape[1]), full(1, b1.shape[1]),
          full(w2.shape[0], d), full(1, d),
      ],
      out_specs=blk(r, d),
      out_shape=jax.ShapeDtypeStruct((n, d), jnp.float32),
  )(p0, p1, x, tbl, w1, b1, w2, b2)


@jax.jit
def kernel(x, edge_index, edge_attr, ee1, ee2, W1, b1, W2, b2):
  n, d = x.shape
  e = edge_index.shape[1]
  n_pad = 10240
  pad = EPAD - e

  src = jnp.pad(edge_index[0], (0, pad)).reshape(NW * CPT, B)
  dst = jnp.pad(edge_index[1], (0, pad), constant_values=n).reshape(NW * CPT, B)
  eidx = jnp.arange(EPAD, dtype=jnp.int32) % 512
  combo = jnp.pad(edge_attr[:, 0] * 3 + edge_attr[:, 1], (0, pad))
  combo = (combo + TROWS * eidx).reshape(NW * CPT, B)

  tbl = _tc_table(ee1, ee2)
  tbl_rep = jnp.tile(tbl, (512, 1))
  (aggr_p,) = _sc_scatter(x, src, dst, combo, tbl_rep, n_pad)

  return _tc_mlp(aggr_p[0, :n], aggr_p[1, :n], x, tbl,
                 W1, b1.reshape(1, -1), W2, b2.reshape(1, -1))


# speedup vs baseline: 6.2132x; 1.0042x over previous
import functools
import jax
import jax.numpy as jnp
from jax import lax
from jax.experimental import pallas as pl
from jax.experimental.pallas import tpu as pltpu
from jax.experimental.pallas import tpu_sc as plsc

NC = 2
NS = 16
NW = NC * NS
B = 128
CPT = 80
EPT = B * CPT
EPAD = NW * EPT
TROWS = 32


def _tbl_body(e1_ref, e2_ref, out_ref):
  c = lax.broadcasted_iota(jnp.int32, (TROWS, 1), 0)
  i6 = lax.broadcasted_iota(jnp.int32, (1, 6), 1)
  i3 = lax.broadcasted_iota(jnp.int32, (1, 3), 1)
  s1 = (c // 3 == i6).astype(jnp.float32)
  s2 = ((c % 3 == i3) & (c < 18)).astype(jnp.float32)
  out_ref[...] = (jnp.dot(s1, e1_ref[...], preferred_element_type=jnp.float32)
                  + jnp.dot(s2, e2_ref[...], preferred_element_type=jnp.float32))


def _tc_table(ee1, ee2):
  d = ee1.shape[1]
  return pl.pallas_call(
      _tbl_body,
      out_shape=jax.ShapeDtypeStruct((TROWS, d), jnp.float32),
  )(ee1, ee2)


def _sc_scatter(x, srcp, dstp, combop, tbl, n_pad):
  D = x.shape[1]
  rpt = n_pad // NS
  mesh = plsc.VectorSubcoreMesh(core_axis_name="c", subcore_axis_name="s")

  @functools.partial(
      pl.kernel,
      out_type=[
          jax.ShapeDtypeStruct((NC, n_pad, D), jnp.float32),
      ],
      mesh=mesh,
      scratch_types=[
          pltpu.VMEM((B,), jnp.int32),
          pltpu.VMEM((B,), jnp.int32),
          pltpu.VMEM((B,), jnp.int32),
          pltpu.VMEM((B,), jnp.int32),
          pltpu.VMEM((B,), jnp.int32),
          pltpu.VMEM((B,), jnp.int32),
          pltpu.VMEM((B, 128), jnp.float32),
          pltpu.VMEM((B, 128), jnp.float32),
          pltpu.VMEM_SHARED((n_pad, 128), jnp.float32),
          pltpu.SemaphoreType.DMA,
          pltpu.SemaphoreType.DMA,
          pltpu.SemaphoreType.DMA,
          pltpu.SemaphoreType.DMA,
      ],
  )
  def k(x_hbm, src_hbm, dst_hbm, combo_hbm, tbl_hbm, aggr_out,
        src_a, src_b, dst_a, dst_b, combo_a, combo_b, rx_v, rt_v, aggr_sh,
        gsem_x, gsem_t, ssem_x, ssem_t):
    c = lax.axis_index("c")
    s = lax.axis_index("s")
    wid = c * NS + s

    def zero_rows(i, carry):
      def zcol(j, carry2):
        rx_v[i, pl.ds(j * 16, 16)] = jnp.zeros((16,), jnp.float32)
        return carry2
      return lax.fori_loop(0, 128 // 16, zcol, carry)
    lax.fori_loop(0, B, zero_rows, 0)

    def zero_shared(j, carry):
      pltpu.sync_copy(rx_v, aggr_sh.at[pl.ds(s * rpt + j * B, B)])
      return carry
    lax.fori_loop(0, rpt // B, zero_shared, 0)

    plsc.subcore_barrier()

    def stage_idx(g, sv, dv, cv):
      base = wid * CPT + g
      pltpu.sync_copy(src_hbm.at[base], sv)
      pltpu.sync_copy(dst_hbm.at[base], dv)
      pltpu.sync_copy(combo_hbm.at[base], cv)

    stage_idx(0, src_a, dst_a, combo_a)
    pltpu.async_copy(x_hbm.at[src_a], rx_v, gsem_x)
    pltpu.async_copy(tbl_hbm.at[combo_a], rt_v, gsem_t)

    def pair(p, carry):
      for q, (sv, dv, cv, sv2, dv2, cv2) in enumerate([
          (src_a, dst_a, combo_a, src_b, dst_b, combo_b),
          (src_b, dst_b, combo_b, src_a, dst_a, combo_a)]):
        g = 2 * p + q
        pltpu.make_async_copy(x_hbm.at[sv], rx_v, gsem_x).wait()
        pltpu.make_async_copy(tbl_hbm.at[cv], rt_v, gsem_t).wait()
        pltpu.async_copy(rx_v, aggr_sh.at[dv], ssem_x, add=True)
        pltpu.async_copy(rt_v, aggr_sh.at[dv], ssem_t, add=True)
        @pl.when(g + 1 < CPT)
        def _():
          stage_idx(g + 1, sv2, dv2, cv2)
        pltpu.make_async_copy(rx_v, aggr_sh.at[dv], ssem_x).wait()
        pltpu.make_async_copy(rt_v, aggr_sh.at[dv], ssem_t).wait()
        @pl.when(g + 1 < CPT)
        def _():
          pltpu.async_copy(x_hbm.at[sv2], rx_v, gsem_x)
          pltpu.async_copy(tbl_hbm.at[cv2], rt_v, gsem_t)
      return carry
    lax.fori_loop(0, CPT // 2, pair, 0)

    plsc.subcore_barrier()

    pltpu.sync_copy(aggr_sh.at[pl.ds(s * rpt, rpt)],
                    aggr_out.at[c, pl.ds(s * rpt, rpt)])

  return k(x, srcp, dstp, combop, tbl)


def _mlp_body(p0, p1, x_ref, tbl, w1, b1, w2, b2, out_ref):
  self_row = tbl[12:13, :]
  aggr = p0[...] + p1[...] + x_ref[...] + self_row
  h = jnp.dot(aggr, w1[...], preferred_element_type=jnp.float32) + b1[...]
  h = jnp.maximum(h, 0.0)
  out_ref[...] = jnp.dot(h, w2[...], preferred_element_type=jnp.float32) + b2[...]


def _tc_mlp(p0, p1, x, tbl, w1, b1, w2, b2):
  n, d = x.shape
  r = 1000
  blk = lambda rr, cc: pl.BlockSpec((rr, cc), lambda i: (i, 0))
  full = lambda rr, cc: pl.BlockSpec((rr, cc), lambda i: (0, 0))
  return pl.pallas_call(
      _mlp_body,
      grid=(n // r,),
      in_specs=[
          blk(r, d), blk(r, d), blk(r, d),
          full(TROWS, d), full(d, w1.shape[1]), full(1, b1.shape[1]),
          full(w2.shape[0], d), full(1, d),
      ],
      out_specs=blk(r, d),
      out_shape=jax.ShapeDtypeStruct((n, d), jnp.float32),
  )(p0, p1, x, tbl, w1, b1, w2, b2)


@jax.jit
def kernel(x, edge_index, edge_attr, ee1, ee2, W1, b1, W2, b2):
  n, d = x.shape
  e = edge_index.shape[1]
  n_pad = 10240
  pad = EPAD - e

  src = jnp.pad(edge_index[0], (0, pad)).reshape(NW * CPT, B)
  dst = jnp.pad(edge_index[1], (0, pad), constant_values=n).reshape(NW * CPT, B)
  ar = jnp.arange(EPAD, dtype=jnp.int32)
  eidx = (ar + (ar // EPT) * 16) % 512
  combo = jnp.pad(edge_attr[:, 0] * 3 + edge_attr[:, 1], (0, pad))
  combo = (combo + TROWS * eidx).reshape(NW * CPT, B)

  tbl = _tc_table(ee1, ee2)
  tbl_rep = jnp.tile(tbl, (512, 1))
  (aggr_p,) = _sc_scatter(x, src, dst, combo, tbl_rep, n_pad)

  return _tc_mlp(aggr_p[0, :n], aggr_p[1, :n], x, tbl,
                 W1, b1.reshape(1, -1), W2, b2.reshape(1, -1))


# speedup vs baseline: 6.2307x; 1.0028x over previous
import functools
import jax
import jax.numpy as jnp
from jax import lax
from jax.experimental import pallas as pl
from jax.experimental.pallas import tpu as pltpu
from jax.experimental.pallas import tpu_sc as plsc

NC = 2
NS = 16
NW = NC * NS
B = 128
CPT = 80
EPT = B * CPT
EPAD = NW * EPT
TROWS = 32


def _tbl_body(e1_ref, e2_ref, out_ref):
  c = lax.broadcasted_iota(jnp.int32, (TROWS, 1), 0)
  i6 = lax.broadcasted_iota(jnp.int32, (1, 6), 1)
  i3 = lax.broadcasted_iota(jnp.int32, (1, 3), 1)
  s1 = (c // 3 == i6).astype(jnp.float32)
  s2 = ((c % 3 == i3) & (c < 18)).astype(jnp.float32)
  out_ref[...] = (jnp.dot(s1, e1_ref[...], preferred_element_type=jnp.float32)
                  + jnp.dot(s2, e2_ref[...], preferred_element_type=jnp.float32))


def _tc_table(ee1, ee2):
  d = ee1.shape[1]
  return pl.pallas_call(
      _tbl_body,
      out_shape=jax.ShapeDtypeStruct((TROWS, d), jnp.float32),
  )(ee1, ee2)


def _sc_scatter(x, srcp, dstp, combop, tbl, n_pad):
  D = x.shape[1]
  rpt = n_pad // NS
  mesh = plsc.VectorSubcoreMesh(core_axis_name="c", subcore_axis_name="s")

  @functools.partial(
      pl.kernel,
      out_type=[
          jax.ShapeDtypeStruct((NC, n_pad, D), jnp.float32),
      ],
      mesh=mesh,
      scratch_types=[
          pltpu.VMEM((B,), jnp.int32),
          pltpu.VMEM((B,), jnp.int32),
          pltpu.VMEM((B,), jnp.int32),
          pltpu.VMEM((B,), jnp.int32),
          pltpu.VMEM((B,), jnp.int32),
          pltpu.VMEM((B,), jnp.int32),
          pltpu.VMEM((B, 128), jnp.float32),
          pltpu.VMEM((B, 128), jnp.float32),
          pltpu.VMEM_SHARED((n_pad, 128), jnp.float32),
          pltpu.SemaphoreType.DMA,
          pltpu.SemaphoreType.DMA,
          pltpu.SemaphoreType.DMA,
          pltpu.SemaphoreType.DMA,
      ],
  )
  def k(x_hbm, src_hbm, dst_hbm, combo_hbm, tbl_hbm, aggr_out,
        src_a, src_b, dst_a, dst_b, combo_a, combo_b, rx_v, rt_v, aggr_sh,
        gsem_x, gsem_t, ssem_x, ssem_t):
    c = lax.axis_index("c")
    s = lax.axis_index("s")
    wid = c * NS + s

    def zero_rows(i, carry):
      def zcol(j, carry2):
        rx_v[i, pl.ds(j * 16, 16)] = jnp.zeros((16,), jnp.float32)
        return carry2
      return lax.fori_loop(0, 128 // 16, zcol, carry)
    lax.fori_loop(0, B, zero_rows, 0)

    def zero_shared(j, carry):
      pltpu.sync_copy(rx_v, aggr_sh.at[pl.ds(s * rpt + j * B, B)])
      return carry
    lax.fori_loop(0, rpt // B, zero_shared, 0)

    plsc.subcore_barrier()

    def stage_idx(g, sv, dv, cv):
      base = wid * CPT + g
      pltpu.sync_copy(src_hbm.at[base], sv)
      pltpu.sync_copy(dst_hbm.at[base], dv)
      pltpu.sync_copy(combo_hbm.at[base], cv)

    stage_idx(0, src_a, dst_a, combo_a)
    pltpu.async_copy(x_hbm.at[src_a], rx_v, gsem_x)
    pltpu.async_copy(tbl_hbm.at[combo_a], rt_v, gsem_t)

    def pair(p, carry):
      for q, (sv, dv, cv, sv2, dv2, cv2) in enumerate([
          (src_a, dst_a, combo_a, src_b, dst_b, combo_b),
          (src_b, dst_b, combo_b, src_a, dst_a, combo_a)]):
        g = 2 * p + q
        pltpu.make_async_copy(x_hbm.at[sv], rx_v, gsem_x).wait()
        pltpu.make_async_copy(tbl_hbm.at[cv], rt_v, gsem_t).wait()
        pltpu.async_copy(rx_v, aggr_sh.at[dv], ssem_x, add=True)
        pltpu.async_copy(rt_v, aggr_sh.at[dv], ssem_t, add=True)
        @pl.when(g + 1 < CPT)
        def _():
          stage_idx(g + 1, sv2, dv2, cv2)
        pltpu.make_async_copy(rx_v, aggr_sh.at[dv], ssem_x).wait()
        pltpu.make_async_copy(rt_v, aggr_sh.at[dv], ssem_t).wait()
        @pl.when(g + 1 < CPT)
        def _():
          pltpu.async_copy(x_hbm.at[sv2], rx_v, gsem_x)
          pltpu.async_copy(tbl_hbm.at[cv2], rt_v, gsem_t)
      return carry
    lax.fori_loop(0, CPT // 2, pair, 0)

    plsc.subcore_barrier()

    pltpu.sync_copy(aggr_sh.at[pl.ds(s * rpt, rpt)],
                    aggr_out.at[c, pl.ds(s * rpt, rpt)])

  return k(x, srcp, dstp, combop, tbl)


def _mlp_body(p0, p1, x_ref, tbl, w1, b1, w2, b2, out_ref):
  self_row = tbl[12:13, :]
  aggr = p0[...] + p1[...] + x_ref[...] + self_row
  h = jnp.dot(aggr, w1[...], preferred_element_type=jnp.float32) + b1[...]
  h = jnp.maximum(h, 0.0)
  out_ref[...] = jnp.dot(h, w2[...], preferred_element_type=jnp.float32) + b2[...]


def _tc_mlp(p0, p1, x, tbl, w1, b1, w2, b2):
  n, d = x.shape
  r = 1000
  blk = lambda rr, cc: pl.BlockSpec((rr, cc), lambda i: (i, 0))
  full = lambda rr, cc: pl.BlockSpec((rr, cc), lambda i: (0, 0))
  return pl.pallas_call(
      _mlp_body,
      grid=(n // r,),
      in_specs=[
          blk(r, d), blk(r, d), blk(r, d),
          full(TROWS, d), full(d, w1.shape[1]), full(1, b1.shape[1]),
          full(w2.shape[0], d), full(1, d),
      ],
      out_specs=blk(r, d),
      out_shape=jax.ShapeDtypeStruct((n, d), jnp.float32),
  )(p0, p1, x, tbl, w1, b1, w2, b2)


@jax.jit
def kernel(x, edge_index, edge_attr, ee1, ee2, W1, b1, W2, b2):
  n, d = x.shape
  e = edge_index.shape[1]
  n_pad = 10240
  pad = EPAD - e

  src = jnp.pad(edge_index[0], (0, pad)).reshape(NW * CPT, B)
  dst = jnp.pad(edge_index[1], (0, pad), constant_values=n).reshape(NW * CPT, B)
  ar = jnp.arange(EPAD, dtype=jnp.int32)
  eidx = (ar + (ar // EPT) * 16) % 512
  combo = jnp.pad(edge_attr[:, 0] * 3 + edge_attr[:, 1], (0, pad))
  combo = (combo * 512 + eidx).reshape(NW * CPT, B)

  tbl = _tc_table(ee1, ee2)
  tbl_rep = jnp.repeat(tbl, 512, axis=0)
  (aggr_p,) = _sc_scatter(x, src, dst, combo, tbl_rep, n_pad)

  return _tc_mlp(aggr_p[0, :n], aggr_p[1, :n], x, tbl,
                 W1, b1.reshape(1, -1), W2, b2.reshape(1, -1))
